# baseline (speedup 1.0000x reference)
import jax
import jax.numpy as jnp
from jax.experimental import pallas as pl
from jax.experimental.pallas import tpu as pltpu

_CHUNK = 1024
_GROUP = 4
_DEPTH = 12


def _mlp_body(s_hbm, a_hbm, w1_ref, b1_ref, w2_ref, b2_ref, w3_ref, b3_ref,
              o_ref, s_buf, a_buf, s_sem, a_sem):
    dim_state = s_hbm.shape[1]
    n_chunks = s_hbm.shape[0] // _CHUNK

    dn = (((1,), (1,)), ((), ()))

    w1s = w1_ref[:, :dim_state].astype(jnp.bfloat16)
    w1a = w1_ref[:, dim_state:].astype(jnp.bfloat16)
    w2 = w2_ref[...].astype(jnp.bfloat16)
    w3 = w3_ref[...].astype(jnp.bfloat16)
    b1 = b1_ref[...]
    b2 = b2_ref[...]
    b3 = b3_ref[0, 0]

    def start(c):
        slot = c % _DEPTH
        pltpu.make_async_copy(s_hbm.at[pl.ds(c * _CHUNK, _CHUNK), :],
                              s_buf.at[slot], s_sem.at[slot]).start()
        pltpu.make_async_copy(a_hbm.at[pl.ds(c * _CHUNK, _CHUNK), :],
                              a_buf.at[slot], a_sem.at[slot]).start()

    def wait(slot):
        pltpu.make_async_copy(s_buf.at[slot], s_buf.at[slot],
                              s_sem.at[slot]).wait()
        pltpu.make_async_copy(a_buf.at[slot], a_buf.at[slot],
                              a_sem.at[slot]).wait()

    n_groups = n_chunks // _GROUP

    def compute(i):
        slot = i % _DEPTH
        s = s_buf[slot].astype(jnp.bfloat16)
        a = a_buf[slot].astype(jnp.bfloat16)

        h = jax.lax.dot_general(s, w1s, dn, preferred_element_type=jnp.float32)
        h += jax.lax.dot_general(a, w1a, dn, preferred_element_type=jnp.float32)
        h = jnp.maximum(h + b1, 0.0).astype(jnp.bfloat16)

        h = jax.lax.dot_general(h, w2, dn, preferred_element_type=jnp.float32)
        h = jnp.maximum(h + b2, 0.0).astype(jnp.bfloat16)

        y = jax.lax.dot_general(w3, h, dn, preferred_element_type=jnp.float32)
        o_ref[:, pl.ds(i * _CHUNK, _CHUNK)] = (y + b3).astype(o_ref.dtype)

    for c in range(2 * _GROUP):
        start(c)
    for g in range(n_groups):
        for c in range(g * _GROUP, (g + 1) * _GROUP):
            wait(c % _DEPTH)
        for c in range(g * _GROUP, (g + 1) * _GROUP):
            compute(c)
        if g + 2 < n_groups:
            for c in range((g + 2) * _GROUP, (g + 3) * _GROUP):
                start(c)


def kernel(state, action, w1, b1, w2, b2, w3, b3):
    batch, dim_state = state.shape
    _, dim_action = action.shape
    hidden, din = w1.shape
    assert batch % (_CHUNK * _GROUP) == 0

    out_shape = jax.ShapeDtypeStruct((1, batch), state.dtype)

    cost = pl.CostEstimate(
        flops=2 * batch * (din * hidden + hidden * hidden + hidden),
        transcendentals=0,
        bytes_accessed=4 * (batch * (din + 1) + hidden * (din + hidden + 3) + 1),
    )

    any_spec = pl.BlockSpec(memory_space=pl.ANY)
    vmem = pl.BlockSpec(memory_space=pltpu.MemorySpace.VMEM)
    smem = pl.BlockSpec(memory_space=pltpu.MemorySpace.SMEM)

    out = pl.pallas_call(
        _mlp_body,
        out_shape=out_shape,
        in_specs=[any_spec, any_spec, vmem, vmem, vmem, vmem, vmem, smem],
        out_specs=vmem,
        scratch_shapes=[
            pltpu.VMEM((_DEPTH, _CHUNK, dim_state), jnp.float32),
            pltpu.VMEM((_DEPTH, _CHUNK, dim_action), jnp.float32),
            pltpu.SemaphoreType.DMA((_DEPTH,)),
            pltpu.SemaphoreType.DMA((_DEPTH,)),
        ],
        cost_estimate=cost,
    )(state, action, w1, b1, w2, b2, w3, b3)
    return out.reshape(batch, 1)


# speedup vs baseline: 1.0558x; 1.0558x over previous
import jax
import jax.numpy as jnp
from jax.experimental import pallas as pl
from jax.experimental.pallas import tpu as pltpu

_NCHUNK = 4


def _mlp_body(s_ref, a_ref, w1s_ref, w1a_ref, b1_ref, w2_ref, b2_ref,
              w3_ref, b3_ref, o_ref):
    dn = (((1,), (1,)), ((), ()))

    w1s = w1s_ref[...].astype(jnp.bfloat16)
    w1a = w1a_ref[...].astype(jnp.bfloat16)
    w2 = w2_ref[...].astype(jnp.bfloat16)
    w3 = w3_ref[...].astype(jnp.bfloat16)
    b1 = b1_ref[...].astype(jnp.bfloat16)
    b2 = b2_ref[...].astype(jnp.bfloat16)
    b3 = b3_ref[0, 0]

    tb = s_ref.shape[0]
    chunk = tb // _NCHUNK

    for c in range(_NCHUNK):
        r0 = c * chunk
        s = s_ref[pl.ds(r0, chunk), :].astype(jnp.bfloat16)
        a = a_ref[pl.ds(r0, chunk), :].astype(jnp.bfloat16)

        h = jax.lax.dot_general(s, w1s, dn, preferred_element_type=jnp.float32)
        h += jax.lax.dot_general(a, w1a, dn, preferred_element_type=jnp.float32)
        h = jnp.maximum(h.astype(jnp.bfloat16) + b1, jnp.bfloat16(0))

        h = jax.lax.dot_general(h, w2, dn, preferred_element_type=jnp.float32)
        h = jnp.maximum(h.astype(jnp.bfloat16) + b2, jnp.bfloat16(0))

        y = jax.lax.dot_general(w3, h, dn, preferred_element_type=jnp.float32)
        o_ref[:, pl.ds(r0, chunk)] = (y + b3).astype(o_ref.dtype)


def kernel(state, action, w1, b1, w2, b2, w3, b3, *, block_batch=4096):
    batch, dim_state = state.shape
    _, dim_action = action.shape
    hidden, din = w1.shape

    out_shape = jax.ShapeDtypeStruct((1, batch), state.dtype)

    cost = pl.CostEstimate(
        flops=2 * batch * (din * hidden + hidden * hidden + hidden),
        transcendentals=0,
        bytes_accessed=4 * (batch * (din + 1) + hidden * (din + hidden + 3) + 1),
    )

    smem = pl.BlockSpec(memory_space=pltpu.MemorySpace.SMEM)

    tb = min(int(block_batch), max(8, 8 * pl.cdiv(pl.cdiv(batch, 4), 8)))
    grid = (pl.cdiv(batch, tb),)

    assert dim_state % dim_action == 0
    a_blk = dim_state // dim_action

    out = pl.pallas_call(
        _mlp_body,
        out_shape=out_shape,
        grid=grid,
        in_specs=[
            pl.BlockSpec((tb, dim_state), lambda i: (i, 0)),
            pl.BlockSpec((tb, dim_action), lambda i: (i, 0)),
            pl.BlockSpec((hidden, dim_state), lambda i: (0, 0)),
            pl.BlockSpec((hidden, dim_action), lambda i: (0, a_blk)),
            pl.BlockSpec((1, hidden), lambda i: (0, 0)),
            pl.BlockSpec((hidden, hidden), lambda i: (0, 0)),
            pl.BlockSpec((1, hidden), lambda i: (0, 0)),
            pl.BlockSpec((1, hidden), lambda i: (0, 0)),
            smem,
        ],
        out_specs=pl.BlockSpec((1, tb), lambda i: (0, i)),
        compiler_params=pltpu.CompilerParams(
            dimension_semantics=("parallel",),
        ),
        cost_estimate=cost,
    )(state, action, w1, w1, b1, w2, b2, w3, b3)
    return out.reshape(batch, 1)


# speedup vs baseline: 1.0598x; 1.0037x over previous
import jax
import jax.numpy as jnp
from jax.experimental import pallas as pl
from jax.experimental.pallas import tpu as pltpu

_NCHUNK = 4


def _mlp_body(s_ref, a_ref, w1s_ref, w1a_ref, b1_ref, w2_ref, b2_ref,
              w3_ref, b3_ref, o_ref):
    dn = (((1,), (1,)), ((), ()))

    w1s = w1s_ref[...].astype(jnp.bfloat16)
    w1a = w1a_ref[...].astype(jnp.bfloat16)
    w2 = w2_ref[...].astype(jnp.bfloat16)
    w3 = w3_ref[...].astype(jnp.bfloat16)
    b1 = b1_ref[...].astype(jnp.bfloat16)
    b2 = b2_ref[...].astype(jnp.bfloat16)
    b3 = b3_ref[0, 0]

    tb = s_ref.shape[0]
    chunk = tb // _NCHUNK
    base = pl.program_id(0) * tb

    for c in range(_NCHUNK):
        r0 = c * chunk
        s = s_ref[pl.ds(r0, chunk), :].astype(jnp.bfloat16)
        a = a_ref[pl.ds(r0, chunk), :].astype(jnp.bfloat16)

        h = jax.lax.dot_general(s, w1s, dn, preferred_element_type=jnp.float32)
        h += jax.lax.dot_general(a, w1a, dn, preferred_element_type=jnp.float32)
        h = jnp.maximum(h.astype(jnp.bfloat16) + b1, jnp.bfloat16(0))

        h = jax.lax.dot_general(h, w2, dn, preferred_element_type=jnp.float32)
        h = jnp.maximum(h.astype(jnp.bfloat16) + b2, jnp.bfloat16(0))

        y = jax.lax.dot_general(w3, h, dn, preferred_element_type=jnp.float32)
        o_ref[:, pl.ds(base + r0, chunk)] = (y + b3).astype(o_ref.dtype)


def kernel(state, action, w1, b1, w2, b2, w3, b3, *, block_batch=4096):
    batch, dim_state = state.shape
    _, dim_action = action.shape
    hidden, din = w1.shape

    out_shape = jax.ShapeDtypeStruct((1, batch), state.dtype)

    cost = pl.CostEstimate(
        flops=2 * batch * (din * hidden + hidden * hidden + hidden),
        transcendentals=0,
        bytes_accessed=4 * (batch * (din + 1) + hidden * (din + hidden + 3) + 1),
    )

    smem = pl.BlockSpec(memory_space=pltpu.MemorySpace.SMEM)

    tb = min(int(block_batch), max(8, 8 * pl.cdiv(pl.cdiv(batch, 4), 8)))
    grid = (pl.cdiv(batch, tb),)

    assert dim_state % dim_action == 0
    a_blk = dim_state // dim_action

    out = pl.pallas_call(
        _mlp_body,
        out_shape=out_shape,
        grid=grid,
        in_specs=[
            pl.BlockSpec((tb, dim_state), lambda i: (i, 0)),
            pl.BlockSpec((tb, dim_action), lambda i: (i, 0)),
            pl.BlockSpec((hidden, dim_state), lambda i: (0, 0)),
            pl.BlockSpec((hidden, dim_action), lambda i: (0, a_blk)),
            pl.BlockSpec((1, hidden), lambda i: (0, 0)),
            pl.BlockSpec((hidden, hidden), lambda i: (0, 0)),
            pl.BlockSpec((1, hidden), lambda i: (0, 0)),
            pl.BlockSpec((1, hidden), lambda i: (0, 0)),
            smem,
        ],
        out_specs=pl.BlockSpec((1, batch), lambda i: (0, 0)),
        compiler_params=pltpu.CompilerParams(
            dimension_semantics=("parallel",),
        ),
        cost_estimate=cost,
    )(state, action, w1, w1, b1, w2, b2, w3, b3)
    return out.reshape(batch, 1)


# speedup vs baseline: 1.0636x; 1.0035x over previous
import jax
import jax.numpy as jnp
from jax.experimental import pallas as pl
from jax.experimental.pallas import tpu as pltpu

_NCHUNK = 8


def _mlp_body(s_ref, a_ref, w1s_ref, w1a_ref, b1_ref, w2_ref, b2_ref,
              w3_ref, b3_ref, o_ref):
    dn = (((1,), (1,)), ((), ()))

    w1s = w1s_ref[...].astype(jnp.bfloat16)
    w1a = w1a_ref[...].astype(jnp.bfloat16)
    w2 = w2_ref[...].astype(jnp.bfloat16)
    w3 = w3_ref[...].astype(jnp.bfloat16)
    b1 = b1_ref[...].astype(jnp.bfloat16)
    b2 = b2_ref[...].astype(jnp.bfloat16)
    b3 = b3_ref[0, 0]

    tb = s_ref.shape[0]
    chunk = tb // _NCHUNK
    base = pl.program_id(0) * tb

    for c in range(_NCHUNK):
        r0 = c * chunk
        s = s_ref[pl.ds(r0, chunk), :].astype(jnp.bfloat16)
        a = a_ref[pl.ds(r0, chunk), :].astype(jnp.bfloat16)

        h = jax.lax.dot_general(s, w1s, dn, preferred_element_type=jnp.float32)
        h += jax.lax.dot_general(a, w1a, dn, preferred_element_type=jnp.float32)
        h = jnp.maximum(h.astype(jnp.bfloat16) + b1, jnp.bfloat16(0))

        h = jax.lax.dot_general(h, w2, dn, preferred_element_type=jnp.float32)
        h = jnp.maximum(h.astype(jnp.bfloat16) + b2, jnp.bfloat16(0))

        y = jax.lax.dot_general(w3, h, dn, preferred_element_type=jnp.float32)
        o_ref[:, pl.ds(base + r0, chunk)] = (y + b3).astype(o_ref.dtype)


def kernel(state, action, w1, b1, w2, b2, w3, b3, *, block_batch=8192):
    batch, dim_state = state.shape
    _, dim_action = action.shape
    hidden, din = w1.shape

    out_shape = jax.ShapeDtypeStruct((1, batch), state.dtype)

    cost = pl.CostEstimate(
        flops=2 * batch * (din * hidden + hidden * hidden + hidden),
        transcendentals=0,
        bytes_accessed=4 * (batch * (din + 1) + hidden * (din + hidden + 3) + 1),
    )

    smem = pl.BlockSpec(memory_space=pltpu.MemorySpace.SMEM)

    tb = min(int(block_batch), max(8, 8 * pl.cdiv(pl.cdiv(batch, 4), 8)))
    grid = (pl.cdiv(batch, tb),)

    assert dim_state % dim_action == 0
    a_blk = dim_state // dim_action

    out = pl.pallas_call(
        _mlp_body,
        out_shape=out_shape,
        grid=grid,
        in_specs=[
            pl.BlockSpec((tb, dim_state), lambda i: (i, 0)),
            pl.BlockSpec((tb, dim_action), lambda i: (i, 0)),
            pl.BlockSpec((hidden, dim_state), lambda i: (0, 0)),
            pl.BlockSpec((hidden, dim_action), lambda i: (0, a_blk)),
            pl.BlockSpec((1, hidden), lambda i: (0, 0)),
            pl.BlockSpec((hidden, hidden), lambda i: (0, 0)),
            pl.BlockSpec((1, hidden), lambda i: (0, 0)),
            pl.BlockSpec((1, hidden), lambda i: (0, 0)),
            smem,
        ],
        out_specs=pl.BlockSpec((1, batch), lambda i: (0, 0)),
        compiler_params=pltpu.CompilerParams(
            dimension_semantics=("parallel",),
        ),
        cost_estimate=cost,
    )(state, action, w1, w1, b1, w2, b2, w3, b3)
    return out.reshape(batch, 1)


# speedup vs baseline: 1.0716x; 1.0076x over previous
import jax
import jax.numpy as jnp
from jax.experimental import pallas as pl
from jax.experimental.pallas import tpu as pltpu

_NCHUNK = 8


def _mlp_body(s_ref, a_ref, w1s_ref, w1a_ref, b1_ref, w2_ref, b2_ref,
              w3_ref, b3_ref, o_ref):
    dn = (((1,), (1,)), ((), ()))

    w1s = w1s_ref[...].astype(jnp.bfloat16)
    w1a = w1a_ref[...].astype(jnp.bfloat16)
    w2 = w2_ref[...].astype(jnp.bfloat16)
    w3 = w3_ref[...].astype(jnp.bfloat16)
    b1 = b1_ref[...].astype(jnp.bfloat16)
    b2 = b2_ref[...].astype(jnp.bfloat16)
    b3 = b3_ref[0, 0]

    tb = s_ref.shape[0]
    chunk = tb // _NCHUNK
    base = pl.program_id(0) * tb

    for c in range(_NCHUNK):
        r0 = c * chunk
        s = s_ref[pl.ds(r0, chunk), :].astype(jnp.bfloat16)
        a = a_ref[pl.ds(r0, chunk), :].astype(jnp.bfloat16)

        h = jax.lax.dot_general(s, w1s, dn, preferred_element_type=jnp.float32)
        h += jax.lax.dot_general(a, w1a, dn, preferred_element_type=jnp.float32)
        h = jnp.maximum(h.astype(jnp.bfloat16) + b1, jnp.bfloat16(0))

        h = jax.lax.dot_general(h, w2, dn, preferred_element_type=jnp.float32)
        h = jnp.maximum(h.astype(jnp.bfloat16) + b2, jnp.bfloat16(0))

        y = jax.lax.dot_general(w3, h, dn, preferred_element_type=jnp.float32)
        o_ref[:, pl.ds(base + r0, chunk)] = (y + b3).astype(o_ref.dtype)


def kernel(state, action, w1, b1, w2, b2, w3, b3, *, block_batch=8192):
    batch, dim_state = state.shape
    _, dim_action = action.shape
    hidden, din = w1.shape

    out_shape = jax.ShapeDtypeStruct((1, batch), state.dtype)

    cost = pl.CostEstimate(
        flops=2 * batch * (din * hidden + hidden * hidden + hidden),
        transcendentals=0,
        bytes_accessed=4 * (batch * (din + 1) + hidden * (din + hidden + 3) + 1),
    )

    smem = pl.BlockSpec(memory_space=pltpu.MemorySpace.SMEM)

    tb = min(int(block_batch), max(8, 8 * pl.cdiv(pl.cdiv(batch, 4), 8)))
    grid = (pl.cdiv(batch, tb),)

    assert dim_state % dim_action == 0
    a_blk = dim_state // dim_action

    out = pl.pallas_call(
        _mlp_body,
        out_shape=out_shape,
        grid=grid,
        in_specs=[
            pl.BlockSpec((tb, dim_state), lambda i: (i, 0)),
            pl.BlockSpec((tb, dim_action), lambda i: (i, 0)),
            pl.BlockSpec((hidden, dim_state), lambda i: (0, 0)),
            pl.BlockSpec((hidden, dim_action), lambda i: (0, a_blk)),
            pl.BlockSpec((1, hidden), lambda i: (0, 0)),
            pl.BlockSpec((hidden, hidden), lambda i: (0, 0)),
            pl.BlockSpec((1, hidden), lambda i: (0, 0)),
            pl.BlockSpec((1, hidden), lambda i: (0, 0)),
            smem,
        ],
        out_specs=pl.BlockSpec((1, batch), lambda i: (0, 0)),
        compiler_params=pltpu.CompilerParams(
            dimension_semantics=("arbitrary",),
        ),
        cost_estimate=cost,
    )(state, action, w1, w1, b1, w2, b2, w3, b3)
    return out.reshape(batch, 1)
